# baseline (speedup 1.0000x reference)
import jax
import jax.numpy as jnp
from jax.experimental import pallas as pl
from jax.experimental.pallas import tpu as pltpu
from jax.experimental.pallas import tpu_sc as plsc

_D = 64
_W = 32


def _gather_rows(pe, idx2d, n):
    mesh = plsc.VectorSubcoreMesh(core_axis_name="c", subcore_axis_name="s")

    @pl.kernel(
        out_type=jax.ShapeDtypeStruct((n, _D), pe.dtype),
        mesh=mesh,
        compiler_params=pltpu.CompilerParams(use_tc_tiling_on_sc=False),
    )
    def gather_kernel(pe_hbm, idx_hbm, out_hbm):
        def body(idx_vmem, out_vmem):
            pltpu.sync_copy(pe_hbm.at[idx_vmem.at[0]], out_vmem)

        pltpu.emit_pipeline(
            body,
            grid=(n // _W,),
            in_specs=[pl.BlockSpec((1, _W), index_map=lambda i: (0, i))],
            out_specs=[pl.BlockSpec((_W, _D), index_map=lambda i: (i, 0))],
            core_axis_name=("c", "s"),
            dimension_semantics=(pltpu.PARALLEL,),
        )(idx_hbm, out_hbm)

    return gather_kernel(pe, idx2d)


def kernel(x, pe):
    b, t = x.shape
    n = b * t
    idx2d = x.reshape(1, n).astype(jnp.int32)
    out = _gather_rows(pe, idx2d, n)
    return out.reshape(b, t, _D)


# speedup vs baseline: 1.1178x; 1.1178x over previous
import jax
import jax.numpy as jnp
from jax import lax
from jax.experimental import pallas as pl
from jax.experimental.pallas import tpu as pltpu
from jax.experimental.pallas import tpu_sc as plsc

_D = 64
_W = 64
_NSUB = 16


def _gather_rows(pe, idx2d, n):
    mesh = plsc.VectorSubcoreMesh(core_axis_name="c", subcore_axis_name="s")
    v = pe.shape[0]
    rows_per_sub = v // _NSUB

    @pl.kernel(
        out_type=jax.ShapeDtypeStruct((n, _D), pe.dtype),
        mesh=mesh,
        scratch_types=[
            pltpu.VMEM_SHARED((v, _D), pe.dtype),
            pltpu.SemaphoreType.DMA,
        ],
        compiler_params=pltpu.CompilerParams(use_tc_tiling_on_sc=False),
    )
    def gather_kernel(pe_hbm, idx_hbm, out_hbm, pe_sh, sem):
        sid = lax.axis_index("s")
        base = sid * rows_per_sub
        pltpu.async_copy(
            pe_hbm.at[pl.ds(base, rows_per_sub)],
            pe_sh.at[pl.ds(base, rows_per_sub)],
            sem,
        ).wait()
        plsc.subcore_barrier()

        def body(idx_vmem, out_vmem):
            pltpu.sync_copy(pe_sh.at[idx_vmem.at[0]], out_vmem)

        pltpu.emit_pipeline(
            body,
            grid=(n // _W,),
            in_specs=[pl.BlockSpec((1, _W), index_map=lambda i: (0, i))],
            out_specs=[pl.BlockSpec((_W, _D), index_map=lambda i: (i, 0))],
            core_axis_name=("c", "s"),
            dimension_semantics=(pltpu.PARALLEL,),
        )(idx_hbm, out_hbm)

    return gather_kernel(pe, idx2d)


def kernel(x, pe):
    b, t = x.shape
    n = b * t
    idx2d = x.reshape(1, n).astype(jnp.int32)
    out = _gather_rows(pe, idx2d, n)
    return out.reshape(b, t, _D)
